# Initial kernel scaffold; baseline (speedup 1.0000x reference)
import jax
import jax.numpy as jnp
from jax.experimental import pallas as pl


def kernel(hidden_states, Wqk, Wv, Wo, rotations):
    raise NotImplementedError("write your pallas kernel here")


# speedup vs baseline: 327.8800x; 327.8800x over previous
import functools

import jax
import jax.numpy as jnp
from jax import lax
from jax.experimental import pallas as pl
from jax.experimental.pallas import tpu as pltpu
from jax.experimental.pallas import tpu_sc as plsc

B = 2
S = 4096
D = 1024
H = 16
DH = 64
CHUNK = 64
NUM_HASHES = 4
NUM_BUCKETS = 128
BH = B * H
M = NUM_HASHES * S
NCH = M // CHUNK
CPB = 8
NB = NCH // CPB
SB = 512
GCHUNK = 512


def _mm_body(x_ref, w_ref, o_ref):
    o_ref[...] = lax.dot_general(
        x_ref[...], w_ref[...], (((1,), (1,)), ((), ())),
        preferred_element_type=jnp.float32)


def _matmul_t(x, w, block_rows=512):
    n, kdim = x.shape
    return pl.pallas_call(
        _mm_body,
        grid=(n // block_rows,),
        in_specs=[
            pl.BlockSpec((block_rows, kdim), lambda i: (i, 0)),
            pl.BlockSpec(w.shape, lambda i: (0, 0)),
        ],
        out_specs=pl.BlockSpec((block_rows, w.shape[0]), lambda i: (i, 0)),
        out_shape=jax.ShapeDtypeStruct((n, w.shape[0]), jnp.float32),
    )(x, w)


def _hash_body(x_ref, r_ref, o_ref):
    x = x_ref[0]
    r = r_ref[0]
    rot = lax.dot_general(x, r, (((1,), (0,)), ((), ())),
                          preferred_element_type=jnp.float32)
    pos = lax.broadcasted_iota(jnp.int32, (S, 1), 0)
    half = NUM_BUCKETS // 2
    for j in range(NUM_HASHES):
        c = rot[:, j * half:(j + 1) * half]
        full = jnp.concatenate([c, -c], axis=1)
        am = jnp.argmax(full, axis=1, keepdims=True).astype(jnp.int32)
        key = S * (am + j * NUM_BUCKETS) + pos
        o_ref[0, j] = key[:, 0]


def _hash_keys(qkh, rot_flat):
    return pl.pallas_call(
        _hash_body,
        grid=(BH,),
        in_specs=[
            pl.BlockSpec((1, S, DH), lambda i: (i, 0, 0)),
            pl.BlockSpec((1,) + rot_flat.shape[1:], lambda i: (i % H, 0, 0)),
        ],
        out_specs=pl.BlockSpec((1, NUM_HASHES, S), lambda i: (i, 0, 0)),
        out_shape=jax.ShapeDtypeStruct((BH, NUM_HASHES, S), jnp.int32),
    )(qkh, rot_flat)


def _sc_gather(table, idx):
    m = idx.shape[0]
    dcols = table.shape[1]
    info = plsc.get_sparse_core_info()
    nw = info.num_cores * info.num_subcores
    rows_per_w = m // nw
    iters = rows_per_w // GCHUNK
    mesh = plsc.VectorSubcoreMesh(core_axis_name="c", subcore_axis_name="s")

    @functools.partial(
        pl.kernel, mesh=mesh,
        out_type=jax.ShapeDtypeStruct((m, dcols), jnp.float32),
        scratch_types=[
            pltpu.VMEM((GCHUNK,), jnp.int32),
            pltpu.VMEM((GCHUNK, dcols), jnp.float32),
            pltpu.SemaphoreType.DMA,
        ],
    )
    def gk(table_hbm, idx_hbm, out_hbm, idx_v, rows_v, sem):
        wid = lax.axis_index("s") * info.num_cores + lax.axis_index("c")
        base0 = wid * rows_per_w

        def body(t, carry):
            base = base0 + t * GCHUNK
            pltpu.sync_copy(idx_hbm.at[pl.ds(base, GCHUNK)], idx_v)
            pltpu.async_copy(table_hbm.at[idx_v], rows_v, sem).wait()
            pltpu.sync_copy(rows_v, out_hbm.at[pl.ds(base, GCHUNK)])
            return carry

        lax.fori_loop(0, iters, body, 0)

    return gk(table, idx)


def _attn_body(cur_ref, prev_ref, o_ref):
    scale = DH ** -0.5
    prev_last = prev_ref[0, CPB - 1]
    for c in range(CPB):
        cur = cur_ref[0, c]
        pch = prev_last if c == 0 else cur_ref[0, c - 1]
        q = cur[:, :DH]
        k = jnp.concatenate([pch[:, :DH], cur[:, :DH]], axis=0) * scale
        v = jnp.concatenate([pch[:, DH:], cur[:, DH:]], axis=0)
        dots = lax.dot_general(q, k, (((1,), (1,)), ((), ())),
                               preferred_element_type=jnp.float32)
        mx = jnp.max(dots, axis=1, keepdims=True)
        ex = jnp.exp(dots - mx)
        ssum = jnp.sum(ex, axis=1, keepdims=True)
        lse = jnp.log(ssum) + mx
        out = lax.dot_general(ex / ssum, v, (((1,), (0,)), ((), ())),
                              preferred_element_type=jnp.float32)
        o_ref[0, c] = jnp.concatenate(
            [out, jnp.broadcast_to(lse, (CHUNK, DH))], axis=1)


def _attention(qv_s):
    return pl.pallas_call(
        _attn_body,
        grid=(BH, NB),
        in_specs=[
            pl.BlockSpec((1, CPB, CHUNK, 2 * DH), lambda i, j: (i, j, 0, 0)),
            pl.BlockSpec((1, CPB, CHUNK, 2 * DH),
                         lambda i, j: (i, (j + NB - 1) % NB, 0, 0)),
        ],
        out_specs=pl.BlockSpec((1, CPB, CHUNK, 2 * DH),
                               lambda i, j: (i, j, 0, 0)),
        out_shape=jax.ShapeDtypeStruct((BH, NCH, CHUNK, 2 * DH), jnp.float32),
    )(qv_s, qv_s)


def _combine_body(u_ref, o_ref):
    xs = [u_ref[0, j] for j in range(NUM_HASHES)]
    m = xs[0][:, DH:DH + 1]
    for x in xs[1:]:
        m = jnp.maximum(m, x[:, DH:DH + 1])
    num = jnp.zeros((SB, DH), jnp.float32)
    den = jnp.zeros((SB, 1), jnp.float32)
    for x in xs:
        w = jnp.exp(x[:, DH:DH + 1] - m)
        num = num + x[:, :DH] * w
        den = den + w
    o_ref[0] = num / den


def _combine(und):
    return pl.pallas_call(
        _combine_body,
        grid=(BH, S // SB),
        in_specs=[pl.BlockSpec((1, NUM_HASHES, SB, 2 * DH),
                               lambda i, j: (i, 0, j, 0))],
        out_specs=pl.BlockSpec((1, SB, DH), lambda i, j: (i, j, 0)),
        out_shape=jax.ShapeDtypeStruct((BH, S, DH), jnp.float32),
    )(und)


def kernel(hidden_states, Wqk, Wv, Wo, rotations):
    hs = hidden_states.reshape(B * S, D)
    qk = _matmul_t(hs, Wqk)
    v = _matmul_t(hs, Wv)
    qkh = qk.reshape(B, S, H, DH).transpose(0, 2, 1, 3).reshape(BH, S, DH)
    vh = v.reshape(B, S, H, DH).transpose(0, 2, 1, 3).reshape(BH, S, DH)

    rot_flat = rotations.reshape(H, DH, NUM_HASHES * (NUM_BUCKETS // 2))
    keys = _hash_keys(qkh, rot_flat).reshape(BH, M)

    sorted_idx = jnp.argsort(keys, axis=-1).astype(jnp.int32)
    undo_idx = jnp.argsort(sorted_idx, axis=-1).astype(jnp.int32)
    per_hash = sorted_idx % S

    qv = jnp.concatenate([qkh, vh], axis=-1).reshape(BH * S, 2 * DH)
    bh_off = (jnp.arange(BH, dtype=jnp.int32) * S)[:, None]
    gidx = (bh_off + per_hash).reshape(BH * M)
    qv_g = _sc_gather(qv, gidx)

    oe = _attention(qv_g.reshape(BH, NCH, CHUNK, 2 * DH))

    bh_off2 = (jnp.arange(BH, dtype=jnp.int32) * M)[:, None]
    uidx = (bh_off2 + undo_idx).reshape(BH * M)
    und = _sc_gather(oe.reshape(BH * M, 2 * DH), uidx)
    und = und.reshape(BH, NUM_HASHES, S, 2 * DH)

    comb = _combine(und)
    y = comb.reshape(B, H, S, DH).transpose(0, 2, 1, 3).reshape(B * S, D)
    return _matmul_t(y, Wo).reshape(B, S, D)


# speedup vs baseline: 573.0354x; 1.7477x over previous
import functools

import jax
import jax.numpy as jnp
from jax import lax
from jax.experimental import pallas as pl
from jax.experimental.pallas import tpu as pltpu
from jax.experimental.pallas import tpu_sc as plsc

B = 2
S = 4096
D = 1024
H = 16
DH = 64
CHUNK = 64
NUM_HASHES = 4
NUM_BUCKETS = 128
BH = B * H
M = NUM_HASHES * S
NCH = M // CHUNK
CPB = 8
NB = NCH // CPB
QR = CPB * CHUNK
KE = QR + CHUNK
OC = 2 * DH
SB = 256
GCHUNK = 512


def _proj_body(x_ref, wq_ref, wv_ref, o_ref, qkh_ref):
    x = x_ref[...]
    qk = lax.dot_general(x, wq_ref[...], (((1,), (1,)), ((), ())),
                         preferred_element_type=jnp.float32)
    v = lax.dot_general(x, wv_ref[...], (((1,), (1,)), ((), ())),
                        preferred_element_type=jnp.float32)
    for h in range(H):
        o_ref[:, h, :DH] = qk[:, h * DH:(h + 1) * DH]
        o_ref[:, h, DH:] = v[:, h * DH:(h + 1) * DH]
        qkh_ref[0, h] = qk[:, h * DH:(h + 1) * DH]


def _proj_qv(hs, wq, wv, block_rows=512):
    n = hs.shape[0]
    nsb = S // block_rows
    return pl.pallas_call(
        _proj_body,
        grid=(n // block_rows,),
        in_specs=[
            pl.BlockSpec((block_rows, D), lambda i: (i, 0)),
            pl.BlockSpec((D, D), lambda i: (0, 0)),
            pl.BlockSpec((D, D), lambda i: (0, 0)),
        ],
        out_specs=[
            pl.BlockSpec((block_rows, H, 2 * DH), lambda i: (i, 0, 0)),
            pl.BlockSpec((1, H, block_rows, DH),
                         lambda i: (i // nsb, 0, i % nsb, 0)),
        ],
        out_shape=[
            jax.ShapeDtypeStruct((n, H, 2 * DH), jnp.float32),
            jax.ShapeDtypeStruct((B, H, S, DH), jnp.float32),
        ],
    )(hs, wq, wv)


def _hash_body(x_ref, r_ref, o_ref):
    x = x_ref[0]
    r = r_ref[0]
    rot = lax.dot_general(x, r, (((1,), (0,)), ((), ())),
                          preferred_element_type=jnp.float32)
    pos = lax.broadcasted_iota(jnp.int32, (S, 1), 0)
    half = NUM_BUCKETS // 2
    for j in range(NUM_HASHES):
        c = rot[:, j * half:(j + 1) * half]
        full = jnp.concatenate([c, -c], axis=1)
        am = jnp.argmax(full, axis=1, keepdims=True).astype(jnp.int32)
        key = S * (am + j * NUM_BUCKETS) + pos
        o_ref[0, j] = key[:, 0]


def _hash_keys(qkh, rot_flat):
    return pl.pallas_call(
        _hash_body,
        grid=(BH,),
        in_specs=[
            pl.BlockSpec((1, S, DH), lambda i: (i, 0, 0)),
            pl.BlockSpec((1,) + rot_flat.shape[1:], lambda i: (i % H, 0, 0)),
        ],
        out_specs=pl.BlockSpec((1, NUM_HASHES, S), lambda i: (i, 0, 0)),
        out_shape=jax.ShapeDtypeStruct((BH, NUM_HASHES, S), jnp.int32),
    )(qkh, rot_flat)


def _sc_gather(table, idx):
    m = idx.shape[0]
    dcols = table.shape[1]
    info = plsc.get_sparse_core_info()
    nw = info.num_cores * info.num_subcores
    rows_per_w = m // nw
    iters = rows_per_w // GCHUNK
    mesh = plsc.VectorSubcoreMesh(core_axis_name="c", subcore_axis_name="s")

    @functools.partial(
        pl.kernel, mesh=mesh,
        out_type=jax.ShapeDtypeStruct((m, dcols), jnp.float32),
        scratch_types=[
            pltpu.VMEM((GCHUNK,), jnp.int32),
            pltpu.VMEM((GCHUNK, dcols), jnp.float32),
            pltpu.SemaphoreType.DMA,
        ],
    )
    def gk(table_hbm, idx_hbm, out_hbm, idx_v, rows_v, sem):
        wid = lax.axis_index("s") * info.num_cores + lax.axis_index("c")
        base0 = wid * rows_per_w

        def body(t, carry):
            base = base0 + t * GCHUNK
            pltpu.sync_copy(idx_hbm.at[pl.ds(base, GCHUNK)], idx_v)
            pltpu.async_copy(table_hbm.at[idx_v], rows_v, sem).wait()
            pltpu.sync_copy(rows_v, out_hbm.at[pl.ds(base, GCHUNK)])
            return carry

        lax.fori_loop(0, iters, body, 0)

    return gk(table, idx)


def _attn_body(cur_ref, prev_ref, o_ref):
    scale = DH ** -0.5
    cur = cur_ref[0].reshape(QR, 2 * DH)
    prevc = prev_ref[0, 0]
    q = cur[:, :DH]
    kv = jnp.concatenate([prevc, cur], axis=0)
    k = kv[:, :DH] * scale
    v = kv[:, DH:]
    dots = lax.dot_general(q, k, (((1,), (1,)), ((), ())),
                           preferred_element_type=jnp.float32)
    rowb = lax.broadcasted_iota(jnp.int32, (QR, KE), 0) // CHUNK
    colb = lax.broadcasted_iota(jnp.int32, (QR, KE), 1) // CHUNK
    valid = (colb == rowb) | (colb == rowb + 1)
    dots = jnp.where(valid, dots, -1e30)
    mx = jnp.max(dots, axis=1, keepdims=True)
    ex = jnp.exp(dots - mx)
    ssum = jnp.sum(ex, axis=1, keepdims=True)
    lse = jnp.log(ssum) + mx
    out = lax.dot_general(ex / ssum, v, (((1,), (0,)), ((), ())),
                          preferred_element_type=jnp.float32)
    packed = jnp.concatenate(
        [out, jnp.broadcast_to(lse, (QR, OC - DH))], axis=1)
    o_ref[0] = packed.reshape(CPB, CHUNK, OC)


def _attention(qv_s):
    return pl.pallas_call(
        _attn_body,
        grid=(BH, NB),
        in_specs=[
            pl.BlockSpec((1, CPB, CHUNK, 2 * DH), lambda i, j: (i, j, 0, 0)),
            pl.BlockSpec((1, 1, CHUNK, 2 * DH),
                         lambda i, j: (i, (j * CPB + NCH - 1) % NCH, 0, 0)),
        ],
        out_specs=pl.BlockSpec((1, CPB, CHUNK, OC), lambda i, j: (i, j, 0, 0)),
        out_shape=jax.ShapeDtypeStruct((BH, NCH, CHUNK, OC), jnp.float32),
    )(qv_s, qv_s)


def _combine_body(u_ref, o_ref):
    for h in range(H):
        xs = [u_ref[h, j] for j in range(NUM_HASHES)]
        m = xs[0][:, DH:DH + 1]
        for x in xs[1:]:
            m = jnp.maximum(m, x[:, DH:DH + 1])
        num = jnp.zeros((SB, DH), jnp.float32)
        den = jnp.zeros((SB, 1), jnp.float32)
        for x in xs:
            w = jnp.exp(x[:, DH:DH + 1] - m)
            num = num + x[:, :DH] * w
            den = den + w
        o_ref[0, :, h * DH:(h + 1) * DH] = num / den


def _combine(und):
    return pl.pallas_call(
        _combine_body,
        grid=(B, S // SB),
        in_specs=[pl.BlockSpec((H, NUM_HASHES, SB, OC),
                               lambda i, j: (i, 0, j, 0))],
        out_specs=pl.BlockSpec((1, SB, D), lambda i, j: (i, j, 0)),
        out_shape=jax.ShapeDtypeStruct((B, S, D), jnp.float32),
    )(und)


def _mm_body(x_ref, w_ref, o_ref):
    o_ref[...] = lax.dot_general(
        x_ref[...], w_ref[...], (((1,), (1,)), ((), ())),
        preferred_element_type=jnp.float32)


def _matmul_t(x, w, block_rows=512):
    n, kdim = x.shape
    return pl.pallas_call(
        _mm_body,
        grid=(n // block_rows,),
        in_specs=[
            pl.BlockSpec((block_rows, kdim), lambda i: (i, 0)),
            pl.BlockSpec(w.shape, lambda i: (0, 0)),
        ],
        out_specs=pl.BlockSpec((block_rows, w.shape[0]), lambda i: (i, 0)),
        out_shape=jax.ShapeDtypeStruct((n, w.shape[0]), jnp.float32),
    )(x, w)


def kernel(hidden_states, Wqk, Wv, Wo, rotations):
    hs = hidden_states.reshape(B * S, D)
    qv, qkh = _proj_qv(hs, Wqk, Wv)

    rot_flat = rotations.reshape(H, DH, NUM_HASHES * (NUM_BUCKETS // 2))
    keys = _hash_keys(qkh.reshape(BH, S, DH), rot_flat).reshape(BH, M)

    sorted_idx = jnp.argsort(keys, axis=-1).astype(jnp.int32)
    undo_idx = jnp.argsort(sorted_idx, axis=-1).astype(jnp.int32)
    per_hash = sorted_idx % S

    bh = jnp.arange(BH, dtype=jnp.int32)
    b_off = ((bh // H) * S)[:, None]
    h_off = (bh % H)[:, None]
    gidx = ((b_off + per_hash) * H + h_off).reshape(BH * M)
    qv_g = _sc_gather(qv.reshape(B * S * H, 2 * DH), gidx)

    oe = _attention(qv_g.reshape(BH, NCH, CHUNK, 2 * DH))

    uidx = ((bh * M)[:, None] + undo_idx).reshape(BH * M)
    und = _sc_gather(oe.reshape(BH * M, OC), uidx)
    und = und.reshape(BH, NUM_HASHES, S, OC)

    y = _combine(und).reshape(B * S, D)
    return _matmul_t(y, Wo).reshape(B, S, D)
